# baseline (speedup 1.0000x reference)
import functools

import jax
import jax.numpy as jnp
from jax import lax
from jax.experimental import pallas as pl
from jax.experimental.pallas import tpu as pltpu
from jax.experimental.pallas import tpu_sc as plsc

N = 10000
E = 320000
D = 128
DE = 16

NC, NS = 1, 16
NW = NC * NS
NCHUNK = 4
E_H = E // NCHUNK
E_PER_W = E_H // NW
BLK_E = 128
N_FULL = E_PER_W // BLK_E
TAIL = E_PER_W - N_FULL * BLK_E
NBUF = 2
N_PAD = 10240
ROWS_PER_TILE = N // NS
ZROWS = 25
IND_R = N_PAD // D

EDGE_BLK = 16000
NODE_BLK = 2000


def _f32_dot(a, w):
    return lax.dot_general(a, w, (((1,), (1,)), ((), ())),
                           preferred_element_type=jnp.float32)


def _edge_mlp_body(e_ref, w1_ref, b1_ref, w2_ref, b2_ref, g_ref, bg_ref,
                   w3_ref, b3_ref, m_ref):
    e = e_ref[...]
    h = lax.dot_general(e, w1_ref[...], (((1,), (1,)), ((), ())),
                        preferred_element_type=jnp.float32)
    h = jnp.maximum(h + b1_ref[...], 0.0)
    h = _f32_dot(h, w2_ref[...]) + b2_ref[...]
    mu = jnp.mean(h, axis=-1, keepdims=True)
    var = jnp.mean((h - mu) * (h - mu), axis=-1, keepdims=True)
    h = (h - mu) * lax.rsqrt(var + 1e-5) * g_ref[...] + bg_ref[...]
    m = _f32_dot(h, w3_ref[...]) + b3_ref[...]
    m_ref[...] = m


def _edge_mlp(edge_attr, w1, b1, w2, b2, g, bg, w3, b3):
    ne = edge_attr.shape[0]
    full = lambda shape: pl.BlockSpec(shape, lambda i: (0, 0))
    return pl.pallas_call(
        _edge_mlp_body,
        grid=(ne // EDGE_BLK,),
        in_specs=[
            pl.BlockSpec((EDGE_BLK, DE), lambda i: (i, 0)),
            full((D, DE)), full((1, D)),
            full((D, D)), full((1, D)),
            full((1, D)), full((1, D)),
            full((D, D)), full((1, D)),
        ],
        out_specs=pl.BlockSpec((EDGE_BLK, D), lambda i: (i, 0)),
        out_shape=jax.ShapeDtypeStruct((ne, D), jnp.float32),
    )(edge_attr, w1, b1, w2, b2, g, bg, w3, b3)


@functools.cache
def _make_sc_scatter():
    return pl.kernel(
        _sc_scatter_body,
        out_type=(jax.ShapeDtypeStruct((N, D), jnp.float32),
                  jax.ShapeDtypeStruct((IND_R, D), jnp.float32)),
        mesh=plsc.VectorSubcoreMesh(core_axis_name="c", subcore_axis_name="s",
                                    num_cores=NC, num_subcores=NS),
        compiler_params=pltpu.CompilerParams(needs_layout_passes=False,
                                             use_tc_tiling_on_sc=False),
        scratch_types=[
            [pltpu.VMEM((BLK_E, D), jnp.float32) for _ in range(NBUF)],
            [pltpu.VMEM((BLK_E,), jnp.int32) for _ in range(NBUF)],
            [pltpu.VMEM((BLK_E,), jnp.int32) for _ in range(NBUF)],
            pltpu.VMEM((TAIL,), jnp.int32),
            pltpu.VMEM((16,), jnp.int32),
            pltpu.VMEM((N_PAD,), jnp.float32),
            pltpu.VMEM((ZROWS, D), jnp.float32),
            pltpu.VMEM((IND_R,), jnp.int32),
            pltpu.VMEM_SHARED((N, D), jnp.float32),
            pltpu.VMEM_SHARED((IND_R, D), jnp.float32),
            [pltpu.SemaphoreType.DMA for _ in range(NBUF)],
            [pltpu.SemaphoreType.DMA for _ in range(NBUF)],
            pltpu.SemaphoreType.DMA,
        ],
    )


def _sc_scatter(m, ei):
    return _make_sc_scatter()(m, ei)


def _sc_scatter_body(m_hbm, ei_hbm, aggr_hbm, ind_hbm,
                     m_bufs, dst_bufs, src_bufs, tdst, tsrc, ind_flat,
                     zbuf, rows_buf, aggr_sh, ind_sh, insems, ssems, zsem):
    sid = lax.axis_index("s")
    ebase = sid * E_PER_W

    zero16 = jnp.zeros((16,), jnp.float32)
    one16 = jnp.ones((16,), jnp.float32)
    iota16 = lax.iota(jnp.int32, 16)

    def fill_z(r, carry):
        for c in range(D // 16):
            zbuf[r, pl.ds(c * 16, 16)] = zero16
        return carry
    lax.fori_loop(0, ZROWS, fill_z, 0)

    def fill_i(r, carry):
        ind_flat[pl.ds(r * 16, 16)] = zero16
        return carry
    lax.fori_loop(0, N_PAD // 16, fill_i, 0)

    for k in range(IND_R // 16):
        rows_buf[pl.ds(k * 16, 16)] = iota16 + k * 16

    row0 = sid * ROWS_PER_TILE
    for k in range(ROWS_PER_TILE // ZROWS):
        pltpu.sync_copy(zbuf, aggr_sh.at[pl.ds(row0 + k * ZROWS, ZROWS)])

    @pl.when(sid == 0)
    def _():
        for k in range(IND_R // ZROWS):
            pltpu.sync_copy(zbuf, ind_sh.at[pl.ds(k * ZROWS, ZROWS)])
        rem = IND_R % ZROWS
        if rem:
            pltpu.sync_copy(zbuf.at[pl.ds(0, rem)],
                            ind_sh.at[pl.ds(IND_R - rem, rem)])
    plsc.subcore_barrier()

    def issue_in(i, j):
        e0 = ebase + i * BLK_E
        pltpu.make_async_copy(
            m_hbm.at[pl.ds(e0, BLK_E)], m_bufs[j], insems[j]).start()
        pltpu.make_async_copy(
            ei_hbm.at[1, pl.ds(e0, BLK_E)], dst_bufs[j], insems[j]).start()
        pltpu.make_async_copy(
            ei_hbm.at[0, pl.ds(e0, BLK_E)], src_bufs[j], insems[j]).start()

    def wait_in(i, j):
        e0 = ebase + i * BLK_E
        pltpu.make_async_copy(
            m_hbm.at[pl.ds(e0, BLK_E)], m_bufs[j], insems[j]).wait()
        pltpu.make_async_copy(
            ei_hbm.at[1, pl.ds(e0, BLK_E)], dst_bufs[j], insems[j]).wait()
        pltpu.make_async_copy(
            ei_hbm.at[0, pl.ds(e0, BLK_E)], src_bufs[j], insems[j]).wait()

    def do_scat(j):
        pltpu.sync_copy(m_bufs[j], aggr_sh.at[dst_bufs[j]], add=True)

    issue_in(0, 0)

    def scat_block(j):
        do_scat(j)
        for k in range(BLK_E // 16):
            sv = src_bufs[j][pl.ds(k * 16, 16)]
            plsc.store_scatter(ind_flat, [sv], one16)

    def outer(t, carry):
        for j in range(NBUF):
            i = t * NBUF + j
            wait_in(i, j)

            @pl.when(i + 1 < N_FULL)
            def _():
                issue_in(i + 1, 1 - j)
            scat_block(j)
        return carry
    lax.fori_loop(0, N_FULL // NBUF, outer, 0)

    for i in range(N_FULL - N_FULL % NBUF, N_FULL):
        wait_in(i, i % NBUF)
        scat_block(i % NBUF)

    if TAIL:
        e0 = ebase + N_FULL * BLK_E
        pltpu.sync_copy(ei_hbm.at[0, pl.ds(e0 - (16 - TAIL), 16)], tsrc)
        pltpu.sync_copy(ei_hbm.at[1, pl.ds(e0, TAIL)], tdst)
        pltpu.sync_copy(m_hbm.at[pl.ds(e0, TAIL)],
                        m_bufs[1].at[pl.ds(0, TAIL)])
        pltpu.sync_copy(m_bufs[1].at[pl.ds(0, TAIL)], aggr_sh.at[tdst],
                        add=True)
        plsc.store_scatter(ind_flat, [tsrc[pl.ds(0, 16)]], one16)

    def pack(r2, carry):
        for c in range(D // 16):
            m_bufs[0][r2, pl.ds(c * 16, 16)] = ind_flat[pl.ds(r2 * D + c * 16, 16)]
        return carry
    lax.fori_loop(0, IND_R, pack, 0)
    pltpu.sync_copy(m_bufs[0].at[pl.ds(0, IND_R)],
                    ind_sh.at[rows_buf], add=True)
    plsc.subcore_barrier()

    pltpu.sync_copy(aggr_sh.at[pl.ds(row0, ROWS_PER_TILE)],
                    aggr_hbm.at[pl.ds(row0, ROWS_PER_TILE)])

    @pl.when(sid == 0)
    def _():
        pltpu.sync_copy(ind_sh, ind_hbm)


def _node_body(*refs):
    x_ref = refs[0]
    a_refs = refs[1:1 + NCHUNK]
    mk_refs = refs[1 + NCHUNK:1 + 2 * NCHUNK]
    (ew_ref, eb_ref, g_ref, bg_ref, esw_ref, esb_ref,
     uwy_ref, uwa_ref, ub_ref) = refs[1 + 2 * NCHUNK:-1]
    o_ref = refs[-1]
    y = _f32_dot(x_ref[...], ew_ref[...]) + eb_ref[...]
    mu = jnp.mean(y, axis=-1, keepdims=True)
    var = jnp.mean((y - mu) * (y - mu), axis=-1, keepdims=True)
    y = (y - mu) * lax.rsqrt(var + 1e-5) * g_ref[...] + bg_ref[...]
    y = _f32_dot(y, esw_ref[...]) + esb_ref[...]
    a = a_refs[0][...]
    mk = mk_refs[0][...]
    for q in range(1, NCHUNK):
        a = a + a_refs[q][...]
        mk = mk + mk_refs[q][...]
    out = (_f32_dot(y, uwy_ref[...]) + _f32_dot(a, uwa_ref[...])
           + ub_ref[...])
    o_ref[...] = jnp.where(mk == 0.0, 0.0, out)


def _node_update(x, aggrs, masks, ew, eb, g, bg, esw, esb, uwy, uwa, ub):
    full = lambda shape: pl.BlockSpec(shape, lambda i: (0, 0))
    blk = lambda shape: pl.BlockSpec(shape, lambda i: (i, 0))
    return pl.pallas_call(
        _node_body,
        grid=(N // NODE_BLK,),
        in_specs=(
            [blk((NODE_BLK, D))]
            + [blk((NODE_BLK, D)) for _ in range(NCHUNK)]
            + [blk((NODE_BLK, 1)) for _ in range(NCHUNK)]
            + [full((D, D)), full((1, D)),
               full((1, D)), full((1, D)),
               full((D, D)), full((1, D)),
               full((D, D)), full((D, D)), full((1, D))]
        ),
        out_specs=pl.BlockSpec((NODE_BLK, D), lambda i: (i, 0)),
        out_shape=jax.ShapeDtypeStruct((N, D), jnp.float32),
    )(x, *aggrs, *masks, ew, eb, g, bg, esw, esb, uwy, uwa, ub)


def kernel(x, edge_index, edge_attr, enc_W, enc_b, ln1_g, ln1_b,
           enc_sup_W, enc_sup_b, msg_W1, msg_b1, msg_W2, msg_b2,
           mnorm_g, mnorm_b, msg_sup_W, msg_sup_b, upd_W, upd_b):
    r = lambda v: v.reshape(1, -1)
    ei = edge_index.astype(jnp.int32)
    mw = (msg_W1, r(msg_b1), msg_W2, r(msg_b2),
          r(mnorm_g), r(mnorm_b), msg_sup_W, r(msg_sup_b))
    aggrs, masks = [], []
    for q in range(NCHUNK):
        mq = _edge_mlp(edge_attr[q * E_H:(q + 1) * E_H], *mw)
        aq, iq = _sc_scatter(mq, ei[:, q * E_H:(q + 1) * E_H])
        aggrs.append(aq)
        masks.append(iq.reshape(N_PAD, 1)[:N])
    out = _node_update(x, aggrs, masks, enc_W, r(enc_b),
                       r(ln1_g), r(ln1_b), enc_sup_W, r(enc_sup_b),
                       upd_W[:, :D], upd_W[:, D:], r(upd_b))
    return out


# speedup vs baseline: 1.0088x; 1.0088x over previous
import functools

import jax
import jax.numpy as jnp
from jax import lax
from jax.experimental import pallas as pl
from jax.experimental.pallas import tpu as pltpu
from jax.experimental.pallas import tpu_sc as plsc

N = 10000
E = 320000
D = 128
DE = 16

NC, NS = 1, 16
NW = NC * NS
E_H = E // 2
E_PER_W = E_H // NW
BLK_E = 128
N_FULL = E_PER_W // BLK_E
TAIL = E_PER_W - N_FULL * BLK_E
NBUF = 2
N_PAD = 10240
ROWS_PER_TILE = N // NS
ZROWS = 25
IND_R = N_PAD // D

EDGE_BLK = 16000
NODE_BLK = 2000


def _f32_dot(a, w):
    return lax.dot_general(a, w, (((1,), (1,)), ((), ())),
                           preferred_element_type=jnp.float32)


def _edge_mlp_body(e_ref, w1_ref, b1_ref, w2_ref, b2_ref, g_ref, bg_ref,
                   w3_ref, b3_ref, m_ref):
    e = e_ref[...]
    h = lax.dot_general(e, w1_ref[...], (((1,), (1,)), ((), ())),
                        preferred_element_type=jnp.float32)
    h = jnp.maximum(h + b1_ref[...], 0.0)
    h = _f32_dot(h, w2_ref[...]) + b2_ref[...]
    mu = jnp.mean(h, axis=-1, keepdims=True)
    var = jnp.mean((h - mu) * (h - mu), axis=-1, keepdims=True)
    h = (h - mu) * lax.rsqrt(var + 1e-5) * g_ref[...] + bg_ref[...]
    m = _f32_dot(h, w3_ref[...]) + b3_ref[...]
    m_ref[...] = m


def _edge_mlp(edge_attr, w1, b1, w2, b2, g, bg, w3, b3):
    ne = edge_attr.shape[0]
    full = lambda shape: pl.BlockSpec(shape, lambda i: (0, 0))
    return pl.pallas_call(
        _edge_mlp_body,
        grid=(ne // EDGE_BLK,),
        in_specs=[
            pl.BlockSpec((EDGE_BLK, DE), lambda i: (i, 0)),
            full((D, DE)), full((1, D)),
            full((D, D)), full((1, D)),
            full((1, D)), full((1, D)),
            full((D, D)), full((1, D)),
        ],
        out_specs=pl.BlockSpec((EDGE_BLK, D), lambda i: (i, 0)),
        out_shape=jax.ShapeDtypeStruct((ne, D), jnp.float32),
    )(edge_attr, w1, b1, w2, b2, g, bg, w3, b3)


@functools.cache
def _make_sc_scatter():
    return pl.kernel(
        _sc_scatter_body,
        out_type=(jax.ShapeDtypeStruct((N, D), jnp.float32),
                  jax.ShapeDtypeStruct((IND_R, D), jnp.float32)),
        mesh=plsc.VectorSubcoreMesh(core_axis_name="c", subcore_axis_name="s",
                                    num_cores=NC, num_subcores=NS),
        compiler_params=pltpu.CompilerParams(needs_layout_passes=False,
                                             use_tc_tiling_on_sc=False),
        scratch_types=[
            [pltpu.VMEM((BLK_E, D), jnp.float32) for _ in range(NBUF)],
            [pltpu.VMEM((BLK_E,), jnp.int32) for _ in range(NBUF)],
            [pltpu.VMEM((BLK_E,), jnp.int32) for _ in range(NBUF)],
            pltpu.VMEM((TAIL,), jnp.int32),
            pltpu.VMEM((TAIL,), jnp.int32),
            pltpu.VMEM((N_PAD,), jnp.float32),
            pltpu.VMEM((ZROWS, D), jnp.float32),
            pltpu.VMEM((IND_R,), jnp.int32),
            pltpu.VMEM_SHARED((N, D), jnp.float32),
            pltpu.VMEM_SHARED((IND_R, D), jnp.float32),
            [pltpu.SemaphoreType.DMA for _ in range(NBUF)],
            [pltpu.SemaphoreType.DMA for _ in range(NBUF)],
            pltpu.SemaphoreType.DMA,
        ],
    )


def _sc_scatter(m, ei):
    return _make_sc_scatter()(m, ei)


def _sc_scatter_body(m_hbm, ei_hbm, aggr_hbm, ind_hbm,
                     m_bufs, dst_bufs, src_bufs, tdst, tsrc, ind_flat,
                     zbuf, rows_buf, aggr_sh, ind_sh, insems, ssems, zsem):
    sid = lax.axis_index("s")
    ebase = sid * E_PER_W

    zero16 = jnp.zeros((16,), jnp.float32)
    one16 = jnp.ones((16,), jnp.float32)
    iota16 = lax.iota(jnp.int32, 16)

    def fill_z(r, carry):
        for c in range(D // 16):
            zbuf[r, pl.ds(c * 16, 16)] = zero16
        return carry
    lax.fori_loop(0, ZROWS, fill_z, 0)

    def fill_i(r, carry):
        ind_flat[pl.ds(r * 16, 16)] = zero16
        return carry
    lax.fori_loop(0, N_PAD // 16, fill_i, 0)

    for k in range(IND_R // 16):
        rows_buf[pl.ds(k * 16, 16)] = iota16 + k * 16

    row0 = sid * ROWS_PER_TILE
    for k in range(ROWS_PER_TILE // ZROWS):
        pltpu.sync_copy(zbuf, aggr_sh.at[pl.ds(row0 + k * ZROWS, ZROWS)])

    @pl.when(sid == 0)
    def _():
        for k in range(IND_R // ZROWS):
            pltpu.sync_copy(zbuf, ind_sh.at[pl.ds(k * ZROWS, ZROWS)])
        rem = IND_R % ZROWS
        if rem:
            pltpu.sync_copy(zbuf.at[pl.ds(0, rem)],
                            ind_sh.at[pl.ds(IND_R - rem, rem)])
    plsc.subcore_barrier()

    def issue_in(i, j):
        e0 = ebase + i * BLK_E
        pltpu.make_async_copy(
            m_hbm.at[pl.ds(e0, BLK_E)], m_bufs[j], insems[j]).start()
        pltpu.make_async_copy(
            ei_hbm.at[1, pl.ds(e0, BLK_E)], dst_bufs[j], insems[j]).start()
        pltpu.make_async_copy(
            ei_hbm.at[0, pl.ds(e0, BLK_E)], src_bufs[j], insems[j]).start()

    def wait_in(i, j):
        e0 = ebase + i * BLK_E
        pltpu.make_async_copy(
            m_hbm.at[pl.ds(e0, BLK_E)], m_bufs[j], insems[j]).wait()
        pltpu.make_async_copy(
            ei_hbm.at[1, pl.ds(e0, BLK_E)], dst_bufs[j], insems[j]).wait()
        pltpu.make_async_copy(
            ei_hbm.at[0, pl.ds(e0, BLK_E)], src_bufs[j], insems[j]).wait()

    def do_scat(j):
        pltpu.sync_copy(m_bufs[j], aggr_sh.at[dst_bufs[j]], add=True)

    issue_in(0, 0)

    def outer(t, carry):
        for j in range(NBUF):
            i = t * NBUF + j
            wait_in(i, j)

            @pl.when(i + 1 < N_FULL)
            def _():
                issue_in(i + 1, 1 - j)
            do_scat(j)
            for k in range(BLK_E // 16):
                sv = src_bufs[j][pl.ds(k * 16, 16)]
                plsc.store_scatter(ind_flat, [sv], one16)
        return carry
    lax.fori_loop(0, N_FULL // NBUF, outer, 0)

    e0 = ebase + N_FULL * BLK_E
    pltpu.sync_copy(ei_hbm.at[0, pl.ds(e0, TAIL)], tsrc)
    pltpu.sync_copy(ei_hbm.at[1, pl.ds(e0, TAIL)], tdst)
    pltpu.sync_copy(m_hbm.at[pl.ds(e0, TAIL)], m_bufs[1].at[pl.ds(0, TAIL)])
    pltpu.sync_copy(m_bufs[1].at[pl.ds(0, TAIL)], aggr_sh.at[tdst], add=True)
    for k in range(TAIL // 16):
        sv = tsrc[pl.ds(k * 16, 16)]
        plsc.store_scatter(ind_flat, [sv], one16)

    def pack(r2, carry):
        for c in range(D // 16):
            m_bufs[0][r2, pl.ds(c * 16, 16)] = ind_flat[pl.ds(r2 * D + c * 16, 16)]
        return carry
    lax.fori_loop(0, IND_R, pack, 0)
    pltpu.sync_copy(m_bufs[0].at[pl.ds(0, IND_R)],
                    ind_sh.at[rows_buf], add=True)
    plsc.subcore_barrier()

    pltpu.sync_copy(aggr_sh.at[pl.ds(row0, ROWS_PER_TILE)],
                    aggr_hbm.at[pl.ds(row0, ROWS_PER_TILE)])

    @pl.when(sid == 0)
    def _():
        pltpu.sync_copy(ind_sh, ind_hbm)


def _node_body(x_ref, a1_ref, a2_ref, mk1_ref, mk2_ref, ew_ref, eb_ref,
               g_ref, bg_ref, esw_ref, esb_ref, uwy_ref, uwa_ref, ub_ref,
               o_ref):
    y = _f32_dot(x_ref[...], ew_ref[...]) + eb_ref[...]
    mu = jnp.mean(y, axis=-1, keepdims=True)
    var = jnp.mean((y - mu) * (y - mu), axis=-1, keepdims=True)
    y = (y - mu) * lax.rsqrt(var + 1e-5) * g_ref[...] + bg_ref[...]
    y = _f32_dot(y, esw_ref[...]) + esb_ref[...]
    a = a1_ref[...] + a2_ref[...]
    out = (_f32_dot(y, uwy_ref[...]) + _f32_dot(a, uwa_ref[...])
           + ub_ref[...])
    mk = mk1_ref[...] + mk2_ref[...]
    o_ref[...] = jnp.where(mk == 0.0, 0.0, out)


def _node_update(x, a1, a2, mk1, mk2, ew, eb, g, bg, esw, esb, uwy, uwa, ub):
    full = lambda shape: pl.BlockSpec(shape, lambda i: (0, 0))
    return pl.pallas_call(
        _node_body,
        grid=(N // NODE_BLK,),
        in_specs=[
            pl.BlockSpec((NODE_BLK, D), lambda i: (i, 0)),
            pl.BlockSpec((NODE_BLK, D), lambda i: (i, 0)),
            pl.BlockSpec((NODE_BLK, D), lambda i: (i, 0)),
            pl.BlockSpec((NODE_BLK, 1), lambda i: (i, 0)),
            pl.BlockSpec((NODE_BLK, 1), lambda i: (i, 0)),
            full((D, D)), full((1, D)),
            full((1, D)), full((1, D)),
            full((D, D)), full((1, D)),
            full((D, D)), full((D, D)), full((1, D)),
        ],
        out_specs=pl.BlockSpec((NODE_BLK, D), lambda i: (i, 0)),
        out_shape=jax.ShapeDtypeStruct((N, D), jnp.float32),
    )(x, a1, a2, mk1, mk2, ew, eb, g, bg, esw, esb, uwy, uwa, ub)


def kernel(x, edge_index, edge_attr, enc_W, enc_b, ln1_g, ln1_b,
           enc_sup_W, enc_sup_b, msg_W1, msg_b1, msg_W2, msg_b2,
           mnorm_g, mnorm_b, msg_sup_W, msg_sup_b, upd_W, upd_b):
    r = lambda v: v.reshape(1, -1)
    ei = edge_index.astype(jnp.int32)
    mw = (msg_W1, r(msg_b1), msg_W2, r(msg_b2),
          r(mnorm_g), r(mnorm_b), msg_sup_W, r(msg_sup_b))
    m1 = _edge_mlp(edge_attr[:E_H], *mw)
    m2 = _edge_mlp(edge_attr[E_H:], *mw)
    aggr1, ind1 = _sc_scatter(m1, ei[:, :E_H])
    aggr2, ind2 = _sc_scatter(m2, ei[:, E_H:])
    ni1 = ind1.reshape(N_PAD, 1)[:N]
    ni2 = ind2.reshape(N_PAD, 1)[:N]
    out = _node_update(x, aggr1, aggr2, ni1, ni2, enc_W, r(enc_b),
                       r(ln1_g), r(ln1_b), enc_sup_W, r(enc_sup_b),
                       upd_W[:, :D], upd_W[:, D:], r(upd_b))
    return out
